# Initial kernel scaffold; baseline (speedup 1.0000x reference)
import functools

import jax
import jax.numpy as jnp
from jax import lax
from jax.experimental import pallas as pl
from jax.experimental.pallas import tpu as pltpu
from jax.experimental.pallas import tpu_sc as plsc

NUM_LEVELS = 16
LOG2_T = 19
TBL = 1 << LOG2_T
N_PTS = 1048576
HIDDEN = 64

_RESOLUTIONS = [16, 20, 25, 32, 40, 50, 64, 80, 101, 128, 161, 202, 256, 322, 406, 512]
_P2 = -1640531535
_P3 = 805459861
_MASK = TBL - 1

NC = 2
NS = 16
NW = NC * NS
PTS_PER_W = N_PTS // NW
CHUNK = 1024
NCHUNKS = PTS_PER_W // CHUNK
NGROUPS = CHUNK // 16


def _encode_body(xp, yp, zp, tab, out, xs, ys, zs, idxv, rows, encv, sem):
    wid = lax.axis_index("s") * NC + lax.axis_index("c")
    base = wid * PTS_PER_W
    iota = lax.iota(jnp.int32, 16)

    def chunk_body(c, carry):
        cbase = base + c * CHUNK
        pltpu.sync_copy(xp.at[pl.ds(cbase, CHUNK)], xs)
        pltpu.sync_copy(yp.at[pl.ds(cbase, CHUNK)], ys)
        pltpu.sync_copy(zp.at[pl.ds(cbase, CHUNK)], zs)

        for l in range(NUM_LEVELS):
            res = float(_RESOLUTIONS[l])
            lvl_off = l * TBL

            def idx_group(g, carry_, res=res, lvl_off=lvl_off):
                s = g * 16
                x = xs[pl.ds(s, 16)] * res
                y = ys[pl.ds(s, 16)] * res
                z = zs[pl.ds(s, 16)] * res
                xi = x.astype(jnp.int32)
                yi = y.astype(jnp.int32)
                zi = z.astype(jnp.int32)
                yh0 = yi * _P2
                yh1 = yh0 + _P2
                zh0 = zi * _P3
                zh1 = zh0 + _P3
                xi1 = xi + 1
                e = (yh0 ^ zh0, yh0 ^ zh1, yh1 ^ zh0, yh1 ^ zh1)
                j = 0
                for xv in (xi, xi1):
                    for yz in range(4):
                        idxv[pl.ds(j * CHUNK + s, 16)] = ((xv ^ e[yz]) & _MASK) + lvl_off
                        j += 1
                return carry_

            lax.fori_loop(0, NGROUPS, idx_group, 0)

            pltpu.async_copy(tab.at[idxv], rows, sem).wait()

            c0 = jnp.zeros((16,), jnp.int32)
            c1 = jnp.ones((16,), jnp.int32)

            def interp_group(g, carry_, l=l, res=res, c0=c0, c1=c1):
                s = g * 16
                x = xs[pl.ds(s, 16)] * res
                y = ys[pl.ds(s, 16)] * res
                z = zs[pl.ds(s, 16)] * res
                wx = x - x.astype(jnp.int32).astype(jnp.float32)
                wy = y - y.astype(jnp.int32).astype(jnp.float32)
                wz = z - z.astype(jnp.int32).astype(jnp.float32)
                rid = s + iota
                for ft, cft in ((0, c0), (1, c1)):
                    f = [plsc.load_gather(rows, [rid + (j * CHUNK), cft])
                         for j in range(8)]
                    c00 = f[0] + wz * (f[1] - f[0])
                    c01 = f[2] + wz * (f[3] - f[2])
                    c10 = f[4] + wz * (f[5] - f[4])
                    c11 = f[6] + wz * (f[7] - f[6])
                    d0 = c00 + wy * (c01 - c00)
                    d1 = c10 + wy * (c11 - c10)
                    v = d0 + wx * (d1 - d0)
                    plsc.store_scatter(encv, [rid, cft + (2 * l)], v)
                return carry_

            lax.fori_loop(0, NGROUPS, interp_group, 0)

        pltpu.sync_copy(encv, out.at[pl.ds(cbase, CHUNK)])
        return carry

    lax.fori_loop(0, NCHUNKS, chunk_body, 0)


def _hash_encode_sc(xp, yp, zp, tab2):
    mesh = plsc.VectorSubcoreMesh(core_axis_name="c", subcore_axis_name="s",
                                  num_cores=NC, num_subcores=NS)
    f = pl.kernel(
        _encode_body,
        out_type=jax.ShapeDtypeStruct((N_PTS, 2 * NUM_LEVELS), jnp.float32),
        mesh=mesh,
        scratch_types=[
            pltpu.VMEM((CHUNK,), jnp.float32),
            pltpu.VMEM((CHUNK,), jnp.float32),
            pltpu.VMEM((CHUNK,), jnp.float32),
            pltpu.VMEM((8 * CHUNK,), jnp.int32),
            pltpu.VMEM((8 * CHUNK, 2), jnp.float32),
            pltpu.VMEM((CHUNK, 2 * NUM_LEVELS), jnp.float32),
            pltpu.SemaphoreType.DMA,
        ],
    )
    return f(xp, yp, zp, tab2)


def _mlp_body(enc_ref, d_ref, w1a_ref, w1b_ref, b1_ref, w2_ref, b2_ref,
              w3_ref, b3_ref, rgb_ref, den_ref):
    d = d_ref[...]
    nrm = jnp.sqrt(jnp.sum(d * d, axis=1, keepdims=True))
    dn = d / jnp.maximum(nrm, 1e-12)
    hp = jax.lax.Precision.HIGHEST
    x = jnp.dot(enc_ref[...], w1a_ref[...], precision=hp,
                preferred_element_type=jnp.float32)
    w1b = w1b_ref[...]
    for i in range(3):
        x = x + dn[:, i:i + 1] * w1b[i:i + 1, :]
    h = jnp.maximum(x + b1_ref[...], 0.0)
    h = jnp.dot(h, w2_ref[...], precision=hp, preferred_element_type=jnp.float32)
    h = jnp.maximum(h + b2_ref[...], 0.0)
    o = jnp.dot(h, w3_ref[...], precision=hp, preferred_element_type=jnp.float32)
    o = o + b3_ref[...]
    rgb_ref[...] = jax.nn.sigmoid(o[:, 0:3])
    den_ref[...] = jnp.maximum(o[:, 3:4], 0.0)


def _mlp_tc(enc, directions, W1, b1, W2, b2, W3, b3):
    B = 8192
    grid = (N_PTS // B,)
    w1a = W1[:32]
    w1b = W1[32:35]
    rep = lambda i: (0, 0)
    return pl.pallas_call(
        _mlp_body,
        grid=grid,
        in_specs=[
            pl.BlockSpec((B, 32), lambda i: (i, 0)),
            pl.BlockSpec((B, 3), lambda i: (i, 0)),
            pl.BlockSpec((32, HIDDEN), rep),
            pl.BlockSpec((3, HIDDEN), rep),
            pl.BlockSpec((1, HIDDEN), rep),
            pl.BlockSpec((HIDDEN, HIDDEN), rep),
            pl.BlockSpec((1, HIDDEN), rep),
            pl.BlockSpec((HIDDEN, 4), rep),
            pl.BlockSpec((1, 4), rep),
        ],
        out_specs=[
            pl.BlockSpec((B, 3), lambda i: (i, 0)),
            pl.BlockSpec((B, 1), lambda i: (i, 0)),
        ],
        out_shape=[
            jax.ShapeDtypeStruct((N_PTS, 3), jnp.float32),
            jax.ShapeDtypeStruct((N_PTS, 1), jnp.float32),
        ],
    )(enc, directions, w1a, w1b, b1.reshape(1, HIDDEN), W2,
      b2.reshape(1, HIDDEN), W3, b3.reshape(1, 4))


def kernel(positions, directions, hash_tables, W1, b1, W2, b2, W3, b3):
    pt = positions.T
    tab2 = hash_tables.reshape(NUM_LEVELS * TBL, 2)
    enc = _hash_encode_sc(pt[0], pt[1], pt[2], tab2)
    rgb, density = _mlp_tc(enc, directions, W1, b1, W2, b2, W3, b3)
    return (rgb, density)


# speedup vs baseline: 71.6458x; 71.6458x over previous
import jax
import jax.numpy as jnp
from jax import lax
from jax.experimental import pallas as pl
from jax.experimental.pallas import tpu as pltpu
from jax.experimental.pallas import tpu_sc as plsc

NUM_LEVELS = 16
LOG2_T = 19
TBL = 1 << LOG2_T
N_PTS = 1048576
HIDDEN = 64
ENC_D = 2 * NUM_LEVELS

_RESOLUTIONS = [16, 20, 25, 32, 40, 50, 64, 80, 101, 128, 161, 203, 256, 322, 406, 512]
_P2 = -1640531535
_P3 = 805459861
_MASK = TBL - 1

NC = 2
NS = 16
NW = NC * NS
PTS_PER_W = N_PTS // NW
CHUNK = 1024
NCHUNKS = PTS_PER_W // CHUNK
NGROUPS = CHUNK // 16


def _encode_body(xp, yp, zp, tab0, tab1, out,
                 xs, ys, zs, idxv, rows0, rows1, encf, sem):
    wid = lax.axis_index("s") * NC + lax.axis_index("c")
    base = wid * PTS_PER_W
    iota = lax.iota(jnp.int32, 16)

    def chunk_body(c, carry):
        cbase = base + c * CHUNK
        pltpu.sync_copy(xp.at[pl.ds(cbase, CHUNK)], xs)
        pltpu.sync_copy(yp.at[pl.ds(cbase, CHUNK)], ys)
        pltpu.sync_copy(zp.at[pl.ds(cbase, CHUNK)], zs)

        for l in range(NUM_LEVELS):
            res = float(_RESOLUTIONS[l])
            lvl_off = l * TBL

            def idx_group(g, carry_, res=res, lvl_off=lvl_off):
                s = g * 16
                x = xs[pl.ds(s, 16)] * res
                y = ys[pl.ds(s, 16)] * res
                z = zs[pl.ds(s, 16)] * res
                xi = x.astype(jnp.int32)
                yi = y.astype(jnp.int32)
                zi = z.astype(jnp.int32)
                yh0 = yi * _P2
                yh1 = yh0 + _P2
                zh0 = zi * _P3
                zh1 = zh0 + _P3
                xi1 = xi + 1
                e = (yh0 ^ zh0, yh0 ^ zh1, yh1 ^ zh0, yh1 ^ zh1)
                j = 0
                for xv in (xi, xi1):
                    for yz in range(4):
                        idxv[pl.ds(j * CHUNK + s, 16)] = ((xv ^ e[yz]) & _MASK) + lvl_off
                        j += 1
                return carry_

            lax.fori_loop(0, NGROUPS, idx_group, 0)

            cp0 = pltpu.async_copy(tab0.at[idxv], rows0, sem)
            cp1 = pltpu.async_copy(tab1.at[idxv], rows1, sem)
            cp0.wait()
            cp1.wait()

            def interp_group(g, carry_, l=l, res=res):
                s = g * 16
                x = xs[pl.ds(s, 16)] * res
                y = ys[pl.ds(s, 16)] * res
                z = zs[pl.ds(s, 16)] * res
                wx = x - x.astype(jnp.int32).astype(jnp.float32)
                wy = y - y.astype(jnp.int32).astype(jnp.float32)
                wz = z - z.astype(jnp.int32).astype(jnp.float32)
                sid = (s + iota) * ENC_D + (2 * l)
                for ft, rows in ((0, rows0), (1, rows1)):
                    f = [rows[pl.ds(j * CHUNK + s, 16)] for j in range(8)]
                    c00 = f[0] + wz * (f[1] - f[0])
                    c01 = f[2] + wz * (f[3] - f[2])
                    c10 = f[4] + wz * (f[5] - f[4])
                    c11 = f[6] + wz * (f[7] - f[6])
                    d0 = c00 + wy * (c01 - c00)
                    d1 = c10 + wy * (c11 - c10)
                    v = d0 + wx * (d1 - d0)
                    plsc.store_scatter(encf, [sid + ft], v)
                return carry_

            lax.fori_loop(0, NGROUPS, interp_group, 0)

        pltpu.sync_copy(encf, out.at[pl.ds(cbase * ENC_D, CHUNK * ENC_D)])
        return carry

    lax.fori_loop(0, NCHUNKS, chunk_body, 0)


def _hash_encode_sc(xp, yp, zp, tab0, tab1):
    mesh = plsc.VectorSubcoreMesh(core_axis_name="c", subcore_axis_name="s",
                                  num_cores=NC, num_subcores=NS)
    f = pl.kernel(
        _encode_body,
        out_type=jax.ShapeDtypeStruct((N_PTS * ENC_D,), jnp.float32),
        mesh=mesh,
        scratch_types=[
            pltpu.VMEM((CHUNK,), jnp.float32),
            pltpu.VMEM((CHUNK,), jnp.float32),
            pltpu.VMEM((CHUNK,), jnp.float32),
            pltpu.VMEM((8 * CHUNK,), jnp.int32),
            pltpu.VMEM((8 * CHUNK,), jnp.float32),
            pltpu.VMEM((8 * CHUNK,), jnp.float32),
            pltpu.VMEM((CHUNK * ENC_D,), jnp.float32),
            pltpu.SemaphoreType.DMA,
        ],
        compiler_params=pltpu.CompilerParams(needs_layout_passes=False),
    )
    return f(xp, yp, zp, tab0, tab1)


def _mlp_body(enc_ref, d_ref, w1_ref, b1_ref, w2_ref, b2_ref,
              w3_ref, b3_ref, rgb_ref, den_ref):
    d = d_ref[...]
    nrm = jnp.sqrt(jnp.sum(d * d, axis=1, keepdims=True))
    dn = d / jnp.maximum(nrm, 1e-12)
    bf = jnp.bfloat16
    mm = lambda a, w: jnp.dot(a.astype(bf), w.astype(bf),
                              preferred_element_type=jnp.float32)
    x = jnp.concatenate([enc_ref[...], dn], axis=-1)
    h = jnp.maximum(mm(x, w1_ref[...]) + b1_ref[...], 0.0)
    h = jnp.maximum(mm(h, w2_ref[...]) + b2_ref[...], 0.0)
    o = mm(h, w3_ref[...]) + b3_ref[...]
    rgb_ref[...] = jax.nn.sigmoid(o[:, 0:3])
    den_ref[...] = jnp.maximum(o[:, 3:4], 0.0)


def _mlp_tc(enc, directions, W1, b1, W2, b2, W3, b3):
    B = 2048
    grid = (N_PTS // B,)
    w1p = jnp.pad(W1, ((0, 5), (0, 0)))
    rep = lambda i: (0, 0)
    return pl.pallas_call(
        _mlp_body,
        grid=grid,
        in_specs=[
            pl.BlockSpec((B, 32), lambda i: (i, 0)),
            pl.BlockSpec((B, 8), lambda i: (i, 0)),
            pl.BlockSpec((40, HIDDEN), rep),
            pl.BlockSpec((1, HIDDEN), rep),
            pl.BlockSpec((HIDDEN, HIDDEN), rep),
            pl.BlockSpec((1, HIDDEN), rep),
            pl.BlockSpec((HIDDEN, 4), rep),
            pl.BlockSpec((1, 4), rep),
        ],
        out_specs=[
            pl.BlockSpec((B, 3), lambda i: (i, 0)),
            pl.BlockSpec((B, 1), lambda i: (i, 0)),
        ],
        out_shape=[
            jax.ShapeDtypeStruct((N_PTS, 3), jnp.float32),
            jax.ShapeDtypeStruct((N_PTS, 1), jnp.float32),
        ],
    )(enc, jnp.pad(directions, ((0, 0), (0, 5))), w1p, b1.reshape(1, HIDDEN),
      W2, b2.reshape(1, HIDDEN), W3, b3.reshape(1, 4))


def kernel(positions, directions, hash_tables, W1, b1, W2, b2, W3, b3):
    pt = positions.T
    tab0 = hash_tables[:, :, 0].reshape(NUM_LEVELS * TBL)
    tab1 = hash_tables[:, :, 1].reshape(NUM_LEVELS * TBL)
    enc = _hash_encode_sc(pt[0], pt[1], pt[2], tab0, tab1)
    enc = enc.reshape(N_PTS, ENC_D)
    rgb, density = _mlp_tc(enc, directions, W1, b1, W2, b2, W3, b3)
    return (rgb, density)
